# Initial kernel scaffold; baseline (speedup 1.0000x reference)
import jax
import jax.numpy as jnp
from jax.experimental import pallas as pl


def kernel(x, edge_index, W1, b1, W2, b2, lpW1, lpb1, lpW2, lpb2):
    raise NotImplementedError("write your pallas kernel here")


# speedup vs baseline: 4.4857x; 4.4857x over previous
import functools

import jax
import jax.numpy as jnp
from jax import lax
from jax.experimental import pallas as pl
from jax.experimental.pallas import tpu as pltpu
from jax.experimental.pallas import tpu_sc as plsc

N = 10000
E = 320000
DIN = 128
DH = 256
DOUT = 128

NC = 2
NS = 16
CHUNK = 128

EP_TILE = ((E // NS + CHUNK - 1) // CHUNK) * CHUNK
N_CHUNKS_16 = EP_TILE // CHUNK
E_PAD1 = EP_TILE * NS

EP_W = ((E // (NC * NS) + CHUNK - 1) // CHUNK) * CHUNK
N_CHUNKS_32 = EP_W // CHUNK
E_PAD2 = EP_W * NC * NS

N_PAD = 10240
ROWS_TILE = N_PAD // NS
ROWS_IO = 128
IO_CHUNKS = ROWS_TILE // ROWS_IO
TRASH = N


_mesh = plsc.VectorSubcoreMesh(core_axis_name="c", subcore_axis_name="s")


def _make_conv_scatter(dh, feature_split):
    n_chunks = N_CHUNKS_16 if feature_split else N_CHUNKS_32

    @functools.partial(
        pl.kernel,
        mesh=_mesh,
        out_type=[jax.ShapeDtypeStruct((N_PAD, dh), jnp.float32)] * 2,
        scratch_types=[
            pltpu.VMEM((CHUNK,), jnp.int32),
            pltpu.VMEM((CHUNK,), jnp.int32),
            pltpu.VMEM((CHUNK, dh), jnp.float32),
            pltpu.VMEM((ROWS_IO, dh), jnp.float32),
            pltpu.VMEM_SHARED((N_PAD, dh), jnp.float32),
            pltpu.SemaphoreType.DMA,
        ],
    )
    def conv_kernel(inL_hbm, inR_hbm, src_hbm, dst_hbm, outL_hbm, outR_hbm,
                    sidx_v, didx_v, rows_v, bounce_v, acc_sh, sem):
        c = lax.axis_index("c")
        s = lax.axis_index("s")

        def run(init_hbm, gat_hbm, out_hbm, ebase):
            for i in range(IO_CHUNKS):
                rb = s * ROWS_TILE + i * ROWS_IO
                pltpu.sync_copy(init_hbm.at[pl.ds(rb, ROWS_IO)], bounce_v)
                pltpu.sync_copy(bounce_v, acc_sh.at[pl.ds(rb, ROWS_IO)])
            plsc.subcore_barrier()

            def body(j, carry):
                off = ebase + j * CHUNK
                pltpu.sync_copy(src_hbm.at[pl.ds(off, CHUNK)], sidx_v)
                pltpu.sync_copy(dst_hbm.at[pl.ds(off, CHUNK)], didx_v)
                pltpu.async_copy(gat_hbm.at[sidx_v], rows_v, sem).wait()
                pltpu.sync_copy(rows_v, acc_sh.at[didx_v], add=True)
                return carry

            lax.fori_loop(0, n_chunks, body, 0)
            plsc.subcore_barrier()
            for i in range(IO_CHUNKS):
                rb = s * ROWS_TILE + i * ROWS_IO
                pltpu.sync_copy(acc_sh.at[pl.ds(rb, ROWS_IO)], bounce_v)
                pltpu.sync_copy(bounce_v, out_hbm.at[pl.ds(rb, ROWS_IO)])

        if feature_split:
            @pl.when(c == 0)
            def _():
                run(inL_hbm, inL_hbm, outL_hbm, s * EP_TILE)

            @pl.when(c == 1)
            def _():
                run(inR_hbm, inR_hbm, outR_hbm, s * EP_TILE)
        else:
            wid = s * NC + c

            @pl.when(c == 0)
            def _():
                run(inL_hbm, inL_hbm, outL_hbm, wid * EP_W)

            @pl.when(c == 1)
            def _():
                run(inR_hbm, inL_hbm, outR_hbm, wid * EP_W)

    return conv_kernel


_conv_scatter_1 = _make_conv_scatter(DH // 2, True)
_conv_scatter_2 = _make_conv_scatter(DOUT, False)


@functools.partial(
    pl.kernel,
    mesh=_mesh,
    out_type=jax.ShapeDtypeStruct((E_PAD2, 16), jnp.float32),
    scratch_types=[
        pltpu.VMEM((CHUNK,), jnp.int32),
        pltpu.VMEM((CHUNK,), jnp.int32),
        pltpu.VMEM((CHUNK, 2 * DOUT), jnp.float32),
        pltpu.VMEM((CHUNK, 2 * DOUT), jnp.float32),
        pltpu.VMEM((16, 16), jnp.float32),
        pltpu.VMEM((CHUNK, 16), jnp.float32),
        pltpu.SemaphoreType.DMA,
    ],
)
def _dec_kernel(a_hbm, b_hbm, src_hbm, dst_hbm, w_hbm, p_hbm,
                sidx_v, didx_v, a_v, b_v, w_v, p_v, sem):
    c = lax.axis_index("c")
    s = lax.axis_index("s")
    wid = s * NC + c
    ebase = wid * EP_W

    pltpu.sync_copy(w_hbm, w_v)
    wregs = [w_v[i, :] for i in range(16)]
    zero = jnp.zeros((16,), jnp.float32)

    def body(j, carry):
        off = ebase + j * CHUNK
        pltpu.sync_copy(src_hbm.at[pl.ds(off, CHUNK)], sidx_v)
        pltpu.sync_copy(dst_hbm.at[pl.ds(off, CHUNK)], didx_v)
        pltpu.async_copy(a_hbm.at[sidx_v], a_v, sem).wait()
        pltpu.async_copy(b_hbm.at[didx_v], b_v, sem).wait()

        def row(r, rc):
            acc = [zero, zero, zero, zero]
            for jj in range(16):
                av = a_v[r, pl.ds(jj * 16, 16)]
                bv = b_v[r, pl.ds(jj * 16, 16)]
                t = jnp.maximum(av + bv, 0.0)
                acc[jj % 4] = t * wregs[jj] + acc[jj % 4]
            p_v[r, :] = (acc[0] + acc[1]) + (acc[2] + acc[3])
            return rc

        lax.fori_loop(0, CHUNK, row, 0)
        pltpu.sync_copy(p_v, p_hbm.at[pl.ds(off, CHUNK)])
        return carry

    lax.fori_loop(0, N_CHUNKS_32, body, 0)


BM = 2048
BE = 2000
NHI = N_PAD // 128


def _deg_body(dst_ref, o_ref):
    i = pl.program_id(0)

    @pl.when(i == 0)
    def _():
        o_ref[...] = jnp.full((NHI, 128), 1.0, jnp.float32)

    d = dst_ref[...]
    hi = lax.broadcasted_iota(jnp.int32, (BE, NHI), 1)
    lo = lax.broadcasted_iota(jnp.int32, (BE, 128), 1)
    u = (d // 128 == hi).astype(jnp.float32)
    v = (d % 128 == lo).astype(jnp.float32)
    o_ref[...] += lax.dot_general(u, v, (((0,), (0,)), ((), ())),
                                  preferred_element_type=jnp.float32)


def _mm1_body(x_ref, w_ref, deg_ref, o_ref):
    dis = lax.rsqrt(deg_ref[...])
    o_ref[...] = dis * jnp.dot(x_ref[...], w_ref[...],
                               preferred_element_type=jnp.float32)


def _mm2_body(s1_ref, deg_ref, b1_ref, w2_ref, o_ref):
    dis = lax.rsqrt(deg_ref[...])
    z1 = jnp.maximum(dis * s1_ref[...] + b1_ref[...], 0.0)
    o_ref[...] = dis * jnp.dot(z1, w2_ref[...],
                               preferred_element_type=jnp.float32)


def _mm3_body(sA_ref, sB_ref, deg_ref, b2_ref, wc_ref, bc_ref, o_ref):
    dis = lax.rsqrt(deg_ref[...])
    z = dis * (sA_ref[...] + sB_ref[...]) + b2_ref[...]
    o_ref[...] = jnp.dot(z, wc_ref[...],
                         preferred_element_type=jnp.float32) + bc_ref[...]


BR = 3200


def _red_body(p_ref, b_ref, o_ref):
    o_ref[...] = jnp.sum(p_ref[...], axis=1, keepdims=True) + b_ref[...]


def kernel(x, edge_index, W1, b1, W2, b2, lpW1, lpb1, lpW2, lpb2):
    src = edge_index[0]
    dst = edge_index[1]

    src1 = jnp.concatenate([src, jnp.zeros((E_PAD1 - E,), jnp.int32)])
    dst1 = jnp.concatenate([dst, jnp.full((E_PAD1 - E,), TRASH, jnp.int32)])
    src2 = jnp.concatenate([src, jnp.zeros((E_PAD2 - E,), jnp.int32)])
    dst2 = jnp.concatenate([dst, jnp.full((E_PAD2 - E,), TRASH, jnp.int32)])
    xp = jnp.pad(x, ((0, N_PAD - N), (0, 0)))
    zerosD = jnp.zeros((N_PAD, DOUT), jnp.float32)

    deg_mat = pl.pallas_call(
        _deg_body,
        grid=(E // BE,),
        in_specs=[pl.BlockSpec((BE, 1), lambda i: (i, 0))],
        out_specs=pl.BlockSpec((NHI, 128), lambda i: (0, 0)),
        out_shape=jax.ShapeDtypeStruct((NHI, 128), jnp.float32),
    )(dst.reshape(E, 1))
    deg_col = deg_mat.reshape(N_PAD, 1)

    h1p = pl.pallas_call(
        _mm1_body,
        grid=(N_PAD // BM,),
        in_specs=[
            pl.BlockSpec((BM, DIN), lambda i: (i, 0)),
            pl.BlockSpec((DIN, DH), lambda i: (0, 0)),
            pl.BlockSpec((BM, 1), lambda i: (i, 0)),
        ],
        out_specs=pl.BlockSpec((BM, DH), lambda i: (i, 0)),
        out_shape=jax.ShapeDtypeStruct((N_PAD, DH), jnp.float32),
    )(xp, W1, deg_col)

    s1L, s1R = _conv_scatter_1(h1p[:, : DH // 2], h1p[:, DH // 2:], src1, dst1)
    s1 = jnp.concatenate([s1L, s1R], axis=1)

    h2p = pl.pallas_call(
        _mm2_body,
        grid=(N_PAD // BM,),
        in_specs=[
            pl.BlockSpec((BM, DH), lambda i: (i, 0)),
            pl.BlockSpec((BM, 1), lambda i: (i, 0)),
            pl.BlockSpec((1, DH), lambda i: (0, 0)),
            pl.BlockSpec((DH, DOUT), lambda i: (0, 0)),
        ],
        out_specs=pl.BlockSpec((BM, DOUT), lambda i: (i, 0)),
        out_shape=jax.ShapeDtypeStruct((N_PAD, DOUT), jnp.float32),
    )(s1, deg_col, b1.reshape(1, DH), W2)

    s2A, s2B = _conv_scatter_2(h2p, zerosD, src2, dst2)

    wcat = jnp.concatenate([lpW1[:DOUT], lpW1[DOUT:]], axis=1)
    bcat = jnp.concatenate([lpb1, jnp.zeros((DH,), jnp.float32)]).reshape(1, -1)

    AB = pl.pallas_call(
        _mm3_body,
        grid=(N_PAD // BM,),
        in_specs=[
            pl.BlockSpec((BM, DOUT), lambda i: (i, 0)),
            pl.BlockSpec((BM, DOUT), lambda i: (i, 0)),
            pl.BlockSpec((BM, 1), lambda i: (i, 0)),
            pl.BlockSpec((1, DOUT), lambda i: (0, 0)),
            pl.BlockSpec((DOUT, 2 * DH), lambda i: (0, 0)),
            pl.BlockSpec((1, 2 * DH), lambda i: (0, 0)),
        ],
        out_specs=pl.BlockSpec((BM, 2 * DH), lambda i: (i, 0)),
        out_shape=jax.ShapeDtypeStruct((N_PAD, 2 * DH), jnp.float32),
    )(s2A, s2B, deg_col, b2.reshape(1, DOUT), wcat, bcat)

    A = AB[:, :DH]
    B = AB[:, DH:]
    w16 = lpW2.reshape(16, 16)

    P = _dec_kernel(A, B, src2, dst2, w16)

    out = pl.pallas_call(
        _red_body,
        grid=(E // BR,),
        in_specs=[
            pl.BlockSpec((BR, 16), lambda i: (i, 0)),
            pl.BlockSpec((1, 1), lambda i: (0, 0)),
        ],
        out_specs=pl.BlockSpec((BR, 1), lambda i: (i, 0)),
        out_shape=jax.ShapeDtypeStruct((E, 1), jnp.float32),
    )(P[:E], lpb2.reshape(1, 1))

    return out
